# baseline (speedup 1.0000x reference)
import jax
import jax.numpy as jnp
from jax import lax
from jax.experimental import pallas as pl
from jax.experimental.pallas import tpu as pltpu
from jax.experimental.pallas import tpu_sc as plsc

N_IN = 10000
D = 256
E = 513
E_PAD = 520
N_OUT = N_IN + E
NC, NS = 2, 16
NW = NC * NS
EPW = 16
LANES = 16
CP_ROWS = 400
CP_N = N_IN // CP_ROWS


def _tail_kernel(x_hbm, if_hbm, t_hbm,
                 idxa_v, idxb_v, bufa, bufb, obuf, sema, semb):
    c = lax.axis_index("c")
    s = lax.axis_index("s")
    wid = s * NC + c

    def batch(fbase, tbase, npairs):
        pltpu.sync_copy(if_hbm.at[pl.ds(fbase, EPW)], idxa_v)
        pltpu.sync_copy(if_hbm.at[pl.ds(fbase + EPW, EPW)], idxb_v)
        hga = pltpu.async_copy(x_hbm.at[idxa_v], bufa, sema)
        hgb = pltpu.async_copy(x_hbm.at[idxb_v], bufb, semb)
        hga.wait()
        hgb.wait()

        def body(e, carry):
            for j in range(D // LANES):
                sl = pl.ds(j * LANES, LANES)
                obuf[e, sl] = (bufa[2 * e, sl] + bufa[2 * e + 1, sl]) * 0.5
                obuf[e + EPW // 2, sl] = (bufb[2 * e, sl]
                                          + bufb[2 * e + 1, sl]) * 0.5
            return carry

        lax.fori_loop(0, EPW // 2, body, 0)
        pltpu.sync_copy(obuf.at[pl.ds(0, npairs)],
                        t_hbm.at[pl.ds(tbase, npairs)])

    batch(wid * 2 * EPW, wid * EPW, EPW)

    @pl.when(wid == NW - 1)
    def _():
        batch(NW * 2 * EPW, NW * EPW, EPW // 2)


def _copy_body(x_hbm, o_hbm, bufs, rsems, wsems):
    hin = [pltpu.make_async_copy(x_hbm.at[pl.ds(c * CP_ROWS, CP_ROWS)],
                                 bufs[c], rsems[c])
           for c in range(CP_N)]
    for h in hin:
        h.start()
    hout = []
    for c in range(CP_N):
        hin[c].wait()
        h = pltpu.make_async_copy(bufs[c],
                                  o_hbm.at[pl.ds(c * CP_ROWS, CP_ROWS)],
                                  wsems[c])
        h.start()
        hout.append(h)
    for h in hout:
        h.wait()


def _insert_body(o0_ref, t_ref, o_ref):
    o_ref[pl.ds(0, E_PAD), :] = t_ref[...]


@jax.jit
def _run(x, idxf):
    mesh = plsc.VectorSubcoreMesh(core_axis_name="c", subcore_axis_name="s",
                                  num_cores=NC, num_subcores=NS)
    t = pl.kernel(
        _tail_kernel,
        out_type=jax.ShapeDtypeStruct((E_PAD, D), jnp.float32),
        mesh=mesh,
        scratch_types=[
            pltpu.VMEM((EPW,), jnp.int32),
            pltpu.VMEM((EPW,), jnp.int32),
            pltpu.VMEM((EPW, D), jnp.float32),
            pltpu.VMEM((EPW, D), jnp.float32),
            pltpu.VMEM((EPW, D), jnp.float32),
            pltpu.SemaphoreType.DMA,
            pltpu.SemaphoreType.DMA,
        ],
    )(x, idxf)

    out0 = pl.pallas_call(
        _copy_body,
        in_specs=[pl.BlockSpec(memory_space=pl.ANY)],
        out_specs=pl.BlockSpec(memory_space=pl.ANY),
        out_shape=jax.ShapeDtypeStruct((N_OUT, D), jnp.float32),
        scratch_shapes=[
            [pltpu.VMEM((CP_ROWS, D), jnp.float32) for _ in range(CP_N)],
            [pltpu.SemaphoreType.DMA for _ in range(CP_N)],
            [pltpu.SemaphoreType.DMA for _ in range(CP_N)],
        ],
    )(x)

    out = pl.pallas_call(
        _insert_body,
        grid=(1,),
        in_specs=[
            pl.BlockSpec((8, 128), lambda i: (0, 0)),
            pl.BlockSpec((E_PAD, D), lambda i: (0, 0)),
        ],
        out_specs=pl.BlockSpec((1000, D), lambda i: (10, 0)),
        out_shape=jax.ShapeDtypeStruct((N_OUT, D), jnp.float32),
        input_output_aliases={0: 0},
    )(out0, t)
    return out


def kernel(input, pool_idx):
    flat = pool_idx.astype(jnp.int32).reshape(-1)
    flat = lax.pad(flat, jnp.int32(0), [(0, 2 * E_PAD + EPW - 2 * E, 0)])
    return _run(input, flat)


# speedup vs baseline: 1.0665x; 1.0665x over previous
import jax
import jax.numpy as jnp
from jax import lax
from jax.experimental import pallas as pl
from jax.experimental.pallas import tpu as pltpu
from jax.experimental.pallas import tpu_sc as plsc

N_IN = 10000
D = 256
E = 513
E_PAD = 520
N_OUT = N_IN + E
NC, NS = 2, 16
NW = NC * NS
EPW = 16
LANES = 16
CP_ROWS = 400
CP_N = N_IN // CP_ROWS


def _tail_kernel(x_hbm, if_hbm, t_hbm,
                 idxa_v, idxb_v, bufa, bufb, obuf, sema, semb):
    c = lax.axis_index("c")
    s = lax.axis_index("s")
    wid = s * NC + c

    def batch(fbase, tbase, both):
        pltpu.sync_copy(if_hbm.at[pl.ds(fbase, EPW)], idxa_v)
        if both:
            pltpu.sync_copy(if_hbm.at[pl.ds(fbase + EPW, EPW)], idxb_v)
        hga = pltpu.async_copy(x_hbm.at[idxa_v], bufa, sema)
        hgb = pltpu.async_copy(x_hbm.at[idxb_v], bufb, semb) if both else None
        hga.wait()
        if both:
            hgb.wait()

        def body(e, carry):
            for j in range(D // LANES):
                sl = pl.ds(j * LANES, LANES)
                obuf[e, sl] = (bufa[2 * e, sl] + bufa[2 * e + 1, sl]) * 0.5
                if both:
                    obuf[e + EPW // 2, sl] = (bufb[2 * e, sl]
                                              + bufb[2 * e + 1, sl]) * 0.5
            return carry

        lax.fori_loop(0, EPW // 2, body, 0)
        npairs = EPW if both else EPW // 2
        pltpu.sync_copy(obuf.at[pl.ds(0, npairs)],
                        t_hbm.at[pl.ds(tbase, npairs)])

    batch(wid * 2 * EPW, wid * EPW, True)

    @pl.when(wid == NW - 1)
    def _():
        batch(NW * 2 * EPW, NW * EPW, False)


def _copy_body(x_hbm, o_hbm, bufs, rsems, wsems):
    hin = [pltpu.make_async_copy(x_hbm.at[pl.ds(c * CP_ROWS, CP_ROWS)],
                                 bufs[c], rsems[c])
           for c in range(CP_N)]
    for h in hin:
        h.start()
    hout = []
    for c in range(CP_N):
        hin[c].wait()
        h = pltpu.make_async_copy(bufs[c],
                                  o_hbm.at[pl.ds(c * CP_ROWS, CP_ROWS)],
                                  wsems[c])
        h.start()
        hout.append(h)
    for h in hout:
        h.wait()


def _insert_body(o0_ref, t_ref, o_ref):
    o_ref[pl.ds(0, E_PAD), :] = t_ref[...]


@jax.jit
def _run(x, idxf):
    mesh = plsc.VectorSubcoreMesh(core_axis_name="c", subcore_axis_name="s",
                                  num_cores=NC, num_subcores=NS)
    t = pl.kernel(
        _tail_kernel,
        out_type=jax.ShapeDtypeStruct((E_PAD, D), jnp.float32),
        mesh=mesh,
        scratch_types=[
            pltpu.VMEM((EPW,), jnp.int32),
            pltpu.VMEM((EPW,), jnp.int32),
            pltpu.VMEM((EPW, D), jnp.float32),
            pltpu.VMEM((EPW, D), jnp.float32),
            pltpu.VMEM((EPW, D), jnp.float32),
            pltpu.SemaphoreType.DMA,
            pltpu.SemaphoreType.DMA,
        ],
    )(x, idxf)

    out0 = pl.pallas_call(
        _copy_body,
        in_specs=[pl.BlockSpec(memory_space=pl.ANY)],
        out_specs=pl.BlockSpec(memory_space=pl.ANY),
        out_shape=jax.ShapeDtypeStruct((N_OUT, D), jnp.float32),
        scratch_shapes=[
            [pltpu.VMEM((CP_ROWS, D), jnp.float32) for _ in range(CP_N)],
            [pltpu.SemaphoreType.DMA for _ in range(CP_N)],
            [pltpu.SemaphoreType.DMA for _ in range(CP_N)],
        ],
    )(x)

    out = pl.pallas_call(
        _insert_body,
        grid=(1,),
        in_specs=[
            pl.BlockSpec((8, 128), lambda i: (0, 0)),
            pl.BlockSpec((E_PAD, D), lambda i: (0, 0)),
        ],
        out_specs=pl.BlockSpec((1000, D), lambda i: (10, 0)),
        out_shape=jax.ShapeDtypeStruct((N_OUT, D), jnp.float32),
        input_output_aliases={0: 0},
    )(out0, t)
    return out


def kernel(input, pool_idx):
    flat = pool_idx.astype(jnp.int32).reshape(-1)
    flat = lax.pad(flat, jnp.int32(0), [(0, 2 * E_PAD - 2 * E, 0)])
    return _run(input, flat)


# speedup vs baseline: 1.1360x; 1.0651x over previous
import jax
import jax.numpy as jnp
from jax import lax
from jax.experimental import pallas as pl
from jax.experimental.pallas import tpu as pltpu
from jax.experimental.pallas import tpu_sc as plsc

N_IN = 10000
D = 256
E = 513
E_PAD = 520
T_ROWS = 1024
N_OUT = N_IN + E
NC, NS = 2, 16
NW = NC * NS
EPW = 16
EPC = 24
LANES = 16
CP_ROWS = 1000
CP_N = N_IN // CP_ROWS


def _tail_kernel(x_hbm, i0_hbm, i1_hbm, t_hbm,
                 idx0_v, idx1_v, buf0, buf1, sem0, sem1):
    c = lax.axis_index("c")
    s = lax.axis_index("s")
    wid = s * NC + c
    ebase = wid * EPW
    pltpu.sync_copy(i0_hbm.at[pl.ds(ebase, EPC)], idx0_v)
    pltpu.sync_copy(i1_hbm.at[pl.ds(ebase, EPC)], idx1_v)
    hg0 = pltpu.async_copy(x_hbm.at[idx0_v], buf0, sem0)
    hg1 = pltpu.async_copy(x_hbm.at[idx1_v], buf1, sem1)
    hg0.wait()
    hg1.wait()

    def body(e, carry):
        for j in range(D // LANES):
            sl = pl.ds(j * LANES, LANES)
            buf0[e, sl] = (buf0[e, sl] + buf1[e, sl]) * 0.5
        return carry

    lax.fori_loop(0, EPC, body, 0)

    pltpu.sync_copy(buf0, t_hbm.at[pl.ds(ebase, EPC)])


def _copy_body(x_hbm, o_hbm, bufs, rsems, wsems):
    hin = [pltpu.make_async_copy(x_hbm.at[pl.ds(c * CP_ROWS, CP_ROWS)],
                                 bufs[c], rsems[c])
           for c in range(CP_N)]
    for h in hin:
        h.start()
    hout = []
    for c in range(CP_N):
        hin[c].wait()
        h = pltpu.make_async_copy(bufs[c],
                                  o_hbm.at[pl.ds(c * CP_ROWS, CP_ROWS)],
                                  wsems[c])
        h.start()
        hout.append(h)
    for h in hout:
        h.wait()


def _insert_body(o0_ref, t_ref, o_ref):
    o_ref[...] = t_ref[...]


@jax.jit
def _run(x, idx0, idx1):
    mesh = plsc.VectorSubcoreMesh(core_axis_name="c", subcore_axis_name="s",
                                  num_cores=NC, num_subcores=NS)
    t = pl.kernel(
        _tail_kernel,
        out_type=jax.ShapeDtypeStruct((T_ROWS, D), jnp.float32),
        mesh=mesh,
        scratch_types=[
            pltpu.VMEM((EPC,), jnp.int32),
            pltpu.VMEM((EPC,), jnp.int32),
            pltpu.VMEM((EPC, D), jnp.float32),
            pltpu.VMEM((EPC, D), jnp.float32),
            pltpu.SemaphoreType.DMA,
            pltpu.SemaphoreType.DMA,
        ],
    )(x, idx0, idx1)

    out0 = pl.pallas_call(
        _copy_body,
        in_specs=[pl.BlockSpec(memory_space=pl.ANY)],
        out_specs=pl.BlockSpec(memory_space=pl.ANY),
        out_shape=jax.ShapeDtypeStruct((N_OUT, D), jnp.float32),
        scratch_shapes=[
            [pltpu.VMEM((CP_ROWS, D), jnp.float32) for _ in range(CP_N)],
            [pltpu.SemaphoreType.DMA for _ in range(CP_N)],
            [pltpu.SemaphoreType.DMA for _ in range(CP_N)],
        ],
    )(x)

    out = pl.pallas_call(
        _insert_body,
        grid=(1,),
        in_specs=[
            pl.BlockSpec((8, 128), lambda i: (0, 0)),
            pl.BlockSpec((CP_ROWS, D), lambda i: (0, 0)),
        ],
        out_specs=pl.BlockSpec((CP_ROWS, D), lambda i: (N_IN // CP_ROWS, 0)),
        out_shape=jax.ShapeDtypeStruct((N_OUT, D), jnp.float32),
        input_output_aliases={0: 0},
    )(out0, t)
    return out


def kernel(input, pool_idx):
    idx = pool_idx.astype(jnp.int32)
    pad = jnp.broadcast_to(idx[-1:], (E_PAD - E, 2))
    idx = jnp.concatenate([idx, pad], axis=0)
    return _run(input, idx[:, 0], idx[:, 1])
